# baseline (speedup 1.0000x reference)
import jax
import jax.numpy as jnp
from jax.experimental import pallas as pl

_BATCH_BLOCK = 64


def _tile_kernel(pos_ref, out_ref):
    out_ref[...] = jnp.broadcast_to(pos_ref[...][None, :, :], out_ref.shape)


def kernel(inputs, pos_embeddings):
    batch, seq, dims = inputs.shape
    pos = pos_embeddings[:seq, :]

    bb = _BATCH_BLOCK
    while batch % bb:
        bb //= 2
    grid = (batch // bb,)

    return pl.pallas_call(
        _tile_kernel,
        grid=grid,
        in_specs=[pl.BlockSpec((seq, dims), lambda i: (0, 0))],
        out_specs=pl.BlockSpec((bb, seq, dims), lambda i: (i, 0, 0)),
        out_shape=jax.ShapeDtypeStruct((batch, seq, dims), pos.dtype),
    )(pos)
